# Initial kernel scaffold; baseline (speedup 1.0000x reference)
import functools

import jax
import jax.numpy as jnp
from jax import lax
from jax.experimental import pallas as pl
from jax.experimental.pallas import tpu as pltpu
from jax.experimental.pallas import tpu_sc as plsc

G = 32
F = 32
NVOX = G * G * G
B = 8
N = 65536
NCHUNK = 8
CN = N // NCHUNK

NC = 2
NS = 16
PTS_PER_TILE = N // NS
SC_CH = 1024
SC_K = PTS_PER_TILE // SC_CH
ROWS_PER_TILE = NVOX // NS
ZR = 512


def _center_body(coords_ref, out_ref):
    c = coords_ref[...]
    out_ref[...] = jnp.mean(c, axis=2, keepdims=True)


def _feat_idx_body(coords_ref, centers_ref, x_ref, wt_ref, b_ref,
                   flat_ref, y_ref):
    c = coords_ref[0]
    ctr = centers_ref[0]
    shifted = (c - ctr) + (G / 2.0)
    idx = jnp.clip(shifted.astype(jnp.int32), 0, G - 1)
    flat_ref[0] = idx[0:1] * (G * G) + idx[1:2] * G + idx[2:3]
    x = x_ref[...]
    y_ref[...] = jnp.dot(x, wt_ref[...],
                         preferred_element_type=jnp.float32) + b_ref[...]


def _sc_scatter_body(feats_hbm, flat_hbm, out_hbm, idx_v, rows_v, zero_v,
                     grid_sh):
    c = lax.axis_index("c")
    s = lax.axis_index("s")

    def fill(i, carry):
        zero_v[i, pl.ds(0, 16)] = jnp.zeros((16,), jnp.float32)
        zero_v[i, pl.ds(16, 16)] = jnp.zeros((16,), jnp.float32)
        return carry
    lax.fori_loop(0, ZR, fill, 0)

    for i in range(B // NC):
        b = c * (B // NC) + i
        for q in range(ROWS_PER_TILE // ZR):
            pltpu.sync_copy(zero_v,
                            grid_sh.at[pl.ds(s * ROWS_PER_TILE + q * ZR, ZR)])
        plsc.subcore_barrier()
        for k in range(SC_K):
            base = b * N + s * PTS_PER_TILE + k * SC_CH
            pltpu.sync_copy(flat_hbm.at[pl.ds(base // 128, SC_CH // 128)],
                            idx_v)
            pltpu.sync_copy(feats_hbm.at[pl.ds(base, SC_CH)], rows_v)
            for j in range(SC_CH // 128):
                pltpu.sync_copy(rows_v.at[pl.ds(j * 128, 128)],
                                grid_sh.at[idx_v.at[j]], add=True)
        plsc.subcore_barrier()
        pltpu.sync_copy(grid_sh.at[pl.ds(s * ROWS_PER_TILE, ROWS_PER_TILE)],
                        out_hbm.at[b, pl.ds(s * ROWS_PER_TILE, ROWS_PER_TILE)])
        plsc.subcore_barrier()


def kernel(coords, features, W, b):
    coords_t = coords.transpose(0, 2, 1)

    centers = pl.pallas_call(
        _center_body,
        out_shape=jax.ShapeDtypeStruct((B, 3, 1), jnp.float32),
    )(coords_t)

    coords_c = (coords_t.reshape(B, 3, NCHUNK, CN)
                .transpose(0, 2, 1, 3).reshape(B * NCHUNK, 3, CN))
    x_flat = features.reshape(B * N, F)

    flat, feats = pl.pallas_call(
        _feat_idx_body,
        grid=(B * NCHUNK,),
        in_specs=[
            pl.BlockSpec((1, 3, CN), lambda j: (j, 0, 0)),
            pl.BlockSpec((1, 3, 1), lambda j: (j // NCHUNK, 0, 0)),
            pl.BlockSpec((CN, F), lambda j: (j, 0)),
            pl.BlockSpec((F, F), lambda j: (0, 0)),
            pl.BlockSpec((1, F), lambda j: (0, 0)),
        ],
        out_specs=[
            pl.BlockSpec((1, 1, CN), lambda j: (j, 0, 0)),
            pl.BlockSpec((CN, F), lambda j: (j, 0)),
        ],
        out_shape=[
            jax.ShapeDtypeStruct((B * NCHUNK, 1, CN), jnp.int32),
            jax.ShapeDtypeStruct((B * N, F), jnp.float32),
        ],
    )(coords_c, centers, x_flat, W.T, b.reshape(1, F))

    flat2 = flat.reshape(B * N // 128, 128)

    mesh = plsc.VectorSubcoreMesh(core_axis_name="c", subcore_axis_name="s")
    grid = pl.kernel(
        _sc_scatter_body,
        mesh=mesh,
        out_type=jax.ShapeDtypeStruct((B, NVOX, F), jnp.float32),
        scratch_types=[
            pltpu.VMEM((SC_CH // 128, 128), jnp.int32),
            pltpu.VMEM((SC_CH, F), jnp.float32),
            pltpu.VMEM((ZR, F), jnp.float32),
            pltpu.VMEM_SHARED((NVOX, F), jnp.float32),
        ],
    )(feats, flat2)

    return grid.reshape(B, G, G, G, F).transpose(0, 4, 1, 2, 3)


# speedup vs baseline: 2.0442x; 2.0442x over previous
import functools

import jax
import jax.numpy as jnp
from jax import lax
from jax.experimental import pallas as pl
from jax.experimental.pallas import tpu as pltpu
from jax.experimental.pallas import tpu_sc as plsc

G = 32
F = 32
NVOX = G * G * G
B = 8
N = 65536
NCHUNK = 8
CN = N // NCHUNK

NC = 2
NS = 16
PTS_PER_TILE = N // NS
SC_CH = 1024
SC_K = PTS_PER_TILE // SC_CH
ROWS_PER_TILE = NVOX // NS
ZR = 512


def _center_body(coords_ref, out_ref):
    c = coords_ref[...]
    out_ref[...] = jnp.mean(c, axis=2, keepdims=True)


def _feat_idx_body(coords_ref, centers_ref, x_ref, wt_ref, b_ref,
                   flat_ref, y_ref):
    c = coords_ref[0]
    ctr = centers_ref[0]
    shifted = (c - ctr) + (G / 2.0)
    idx = jnp.clip(shifted.astype(jnp.int32), 0, G - 1)
    flat_ref[0] = idx[0:1] * (G * G) + idx[1:2] * G + idx[2:3]
    x = x_ref[...]
    y_ref[...] = jnp.dot(x, wt_ref[...],
                         preferred_element_type=jnp.float32) + b_ref[...]


def _sc_scatter_body(feats_hbm, flat_hbm, out_hbm, idx_v, rows_v, zero_v,
                     grid_sh):
    c = lax.axis_index("c")
    s = lax.axis_index("s")

    def fill(i, carry):
        zero_v[i, pl.ds(0, 16)] = jnp.zeros((16,), jnp.float32)
        zero_v[i, pl.ds(16, 16)] = jnp.zeros((16,), jnp.float32)
        return carry
    lax.fori_loop(0, ZR, fill, 0)

    row0 = pl.multiple_of(s * ROWS_PER_TILE, 8)
    for i in range(B // NC):
        b = c * (B // NC) + i
        for q in range(ROWS_PER_TILE // ZR):
            pltpu.sync_copy(zero_v, grid_sh.at[pl.ds(row0 + q * ZR, ZR)])
        plsc.subcore_barrier()
        for k in range(SC_K):
            base = pl.multiple_of(b * N + s * PTS_PER_TILE + k * SC_CH, 1024)
            pltpu.sync_copy(
                flat_hbm.at[pl.ds(pl.multiple_of(base // 128, 8),
                                  SC_CH // 128)],
                idx_v)
            pltpu.sync_copy(feats_hbm.at[pl.ds(base, SC_CH)], rows_v)
            for j in range(SC_CH // 128):
                pltpu.sync_copy(rows_v.at[pl.ds(j * 128, 128)],
                                grid_sh.at[idx_v.at[j]], add=True)
        plsc.subcore_barrier()
        pltpu.sync_copy(grid_sh.at[pl.ds(row0, ROWS_PER_TILE)],
                        out_hbm.at[b, pl.ds(row0, ROWS_PER_TILE)])
        plsc.subcore_barrier()


def kernel(coords, features, W, b):
    coords_t = coords.transpose(0, 2, 1)

    centers = pl.pallas_call(
        _center_body,
        out_shape=jax.ShapeDtypeStruct((B, 3, 1), jnp.float32),
    )(coords_t)

    coords_c = (coords_t.reshape(B, 3, NCHUNK, CN)
                .transpose(0, 2, 1, 3).reshape(B * NCHUNK, 3, CN))
    x_flat = features.reshape(B * N, F)

    flat, feats = pl.pallas_call(
        _feat_idx_body,
        grid=(B * NCHUNK,),
        in_specs=[
            pl.BlockSpec((1, 3, CN), lambda j: (j, 0, 0)),
            pl.BlockSpec((1, 3, 1), lambda j: (j // NCHUNK, 0, 0)),
            pl.BlockSpec((CN, F), lambda j: (j, 0)),
            pl.BlockSpec((F, F), lambda j: (0, 0)),
            pl.BlockSpec((1, F), lambda j: (0, 0)),
        ],
        out_specs=[
            pl.BlockSpec((1, 1, CN), lambda j: (j, 0, 0)),
            pl.BlockSpec((CN, F), lambda j: (j, 0)),
        ],
        out_shape=[
            jax.ShapeDtypeStruct((B * NCHUNK, 1, CN), jnp.int32),
            jax.ShapeDtypeStruct((B * N, F), jnp.float32),
        ],
    )(coords_c, centers, x_flat, W.T, b.reshape(1, F))

    flat2 = flat.reshape(B * N // 128, 128)

    mesh = plsc.VectorSubcoreMesh(core_axis_name="c", subcore_axis_name="s")
    grid = pl.kernel(
        _sc_scatter_body,
        mesh=mesh,
        compiler_params=pltpu.CompilerParams(use_tc_tiling_on_sc=False),
        out_type=jax.ShapeDtypeStruct((B, NVOX, F), jnp.float32),
        scratch_types=[
            pltpu.VMEM((SC_CH // 128, 128), jnp.int32),
            pltpu.VMEM((SC_CH, F), jnp.float32),
            pltpu.VMEM((ZR, F), jnp.float32),
            pltpu.VMEM_SHARED((NVOX, F), jnp.float32),
        ],
    )(feats, flat2)

    return grid.reshape(B, G, G, G, F).transpose(0, 4, 1, 2, 3)


# speedup vs baseline: 2.2637x; 1.1074x over previous
import functools

import jax
import jax.numpy as jnp
from jax import lax
from jax.experimental import pallas as pl
from jax.experimental.pallas import tpu as pltpu
from jax.experimental.pallas import tpu_sc as plsc

G = 32
F = 32
NVOX = G * G * G
B = 8
N = 65536
NCHUNK = 8
CN = N // NCHUNK

NC = 2
NS = 16
PLANES = F + 1
TASKS = (B // NC) * PLANES
KMAX = -(-TASKS // NS)
PCH = 32768
PROWS = PCH // CN
NCH = N // PCH
UNROLL = 8


def _center_body(coords_ref, out_ref):
    c = coords_ref[...]
    out_ref[...] = jnp.mean(c, axis=2)


def _index_body(coords_ref, centers_ref, flat_ref):
    c = coords_ref[...]
    ctr = centers_ref[...][:, :, None]
    shifted = (c - ctr) + (G / 2.0)
    idx = jnp.clip(shifted.astype(jnp.int32), 0, G - 1)
    fl = idx[0] * (G * G) + idx[1] * G + idx[2]
    flat_ref[0] = fl.astype(jnp.float32)


def _final_body(g_ref, cnt_ref, w_ref, b_ref, out_ref):
    g = g_ref[0]
    y = jnp.dot(w_ref[...], g, preferred_element_type=jnp.float32)
    out_ref[0] = y + b_ref[...] * cnt_ref[0]


def _sc_body(vals_hbm, idx_hbm, grid_hbm, cnt_hbm, idx_v, val_v, plane_v):
    c = lax.axis_index("c")
    s = lax.axis_index("s")

    zero16 = jnp.zeros((16,), jnp.float32)
    one16 = jnp.ones((16,), jnp.float32)

    for k in range(KMAX):
        t = s + NS * k

        @pl.when(t < TASKS)
        def _task():
            bi = t // PLANES
            plane = t - bi * PLANES
            b = c * (B // NC) + bi

            def zero_body(i, carry):
                for u in range(4):
                    plane_v[pl.ds((i * 4 + u) * 16, 16)] = zero16
                return carry
            lax.fori_loop(0, NVOX // 64, zero_body, 0)

            @pl.when(plane == F)
            def _fill_ones():
                def ones_body(i, carry):
                    for u in range(4):
                        val_v[pl.ds((i * 4 + u) * 16, 16)] = one16
                    return carry
                lax.fori_loop(0, PCH // 64, ones_body, 0)

            for ch in range(NCH):
                for r in range(PROWS):
                    pltpu.sync_copy(idx_hbm.at[ch * PROWS + r, b],
                                    idx_v.at[r])
                @pl.when(plane < F)
                def _load_vals():
                    pltpu.sync_copy(
                        vals_hbm.at[b, plane, pl.ds(ch * PCH, PCH)], val_v)

                for r in range(PROWS):
                    def grp(i, carry):
                        for u in range(UNROLL):
                            off = (i * UNROLL + u) * 16
                            idx16 = idx_v[r, pl.ds(off, 16)]
                            val16 = val_v[pl.ds(r * CN + off, 16)]
                            plsc.addupdate_scatter(plane_v, [idx16], val16)
                        return carry
                    lax.fori_loop(0, CN // (16 * UNROLL), grp, 0)

            @pl.when(plane < F)
            def _store_plane():
                pltpu.sync_copy(plane_v, grid_hbm.at[b, plane])
            @pl.when(plane == F)
            def _store_counts():
                pltpu.sync_copy(plane_v, cnt_hbm.at[b, 0])


def kernel(coords, features, W, b):
    coords_p = coords.transpose(2, 0, 1)
    vals = features.transpose(0, 2, 1)

    centers = pl.pallas_call(
        _center_body,
        out_shape=jax.ShapeDtypeStruct((3, B), jnp.float32),
    )(coords_p)

    flatf = pl.pallas_call(
        _index_body,
        grid=(NCHUNK,),
        in_specs=[
            pl.BlockSpec((3, B, CN), lambda j: (0, 0, j)),
            pl.BlockSpec((3, B), lambda j: (0, 0)),
        ],
        out_specs=pl.BlockSpec((1, B, CN), lambda j: (j, 0, 0)),
        out_shape=jax.ShapeDtypeStruct((NCHUNK, B, CN), jnp.float32),
    )(coords_p, centers)
    flat = flatf.astype(jnp.int32)

    mesh = plsc.VectorSubcoreMesh(core_axis_name="c", subcore_axis_name="s")
    grid0, counts = pl.kernel(
        _sc_body,
        mesh=mesh,
        compiler_params=pltpu.CompilerParams(use_tc_tiling_on_sc=False,
                                             needs_layout_passes=False),
        out_type=[
            jax.ShapeDtypeStruct((B, F, NVOX), jnp.float32),
            jax.ShapeDtypeStruct((B, 1, NVOX), jnp.float32),
        ],
        scratch_types=[
            pltpu.VMEM((PROWS, CN), jnp.int32),
            pltpu.VMEM((PCH,), jnp.float32),
            pltpu.VMEM((NVOX,), jnp.float32),
        ],
    )(vals, flat)

    out = pl.pallas_call(
        _final_body,
        grid=(B, NVOX // 2048),
        in_specs=[
            pl.BlockSpec((1, F, 2048), lambda i, j: (i, 0, j)),
            pl.BlockSpec((1, 1, 2048), lambda i, j: (i, 0, j)),
            pl.BlockSpec((F, F), lambda i, j: (0, 0)),
            pl.BlockSpec((F, 1), lambda i, j: (0, 0)),
        ],
        out_specs=pl.BlockSpec((1, F, 2048), lambda i, j: (i, 0, j)),
        out_shape=jax.ShapeDtypeStruct((B, F, NVOX), jnp.float32),
    )(grid0, counts, W, b.reshape(F, 1))

    return out.reshape(B, F, G, G, G)
